# baseline (speedup 1.0000x reference)
import functools

import jax
import jax.numpy as jnp
from jax import lax
from jax.experimental import pallas as pl
from jax.experimental.pallas import tpu as pltpu
from jax.experimental.pallas import tpu_sc as plsc

D_MODEL = 1024
NUM_IDX = 4 * 8192

NUM_CORES = 2
NUM_SUBCORES = 16
NUM_WORKERS = NUM_CORES * NUM_SUBCORES
PER_WORKER = NUM_IDX // NUM_WORKERS
CHUNK = 32
NUM_CHUNKS = PER_WORKER // CHUNK
NBUF = 3
NUM_GROUPS = 10

_mesh = plsc.VectorSubcoreMesh(core_axis_name="c", subcore_axis_name="s")


@functools.partial(
    pl.kernel,
    mesh=_mesh,
    out_type=jax.ShapeDtypeStruct((NUM_IDX, D_MODEL), jnp.float32),
    scratch_types=[
        pltpu.VMEM((PER_WORKER,), jnp.int32),
        pltpu.VMEM((CHUNK, D_MODEL), jnp.float32),
        pltpu.VMEM((CHUNK, D_MODEL), jnp.float32),
        pltpu.VMEM((CHUNK, D_MODEL), jnp.float32),
        pltpu.SemaphoreType.DMA,
        pltpu.SemaphoreType.DMA,
        pltpu.SemaphoreType.DMA,
        pltpu.SemaphoreType.DMA,
        pltpu.SemaphoreType.DMA,
        pltpu.SemaphoreType.DMA,
    ],
)
def _gather_rows(
    idx_hbm, table_hbm, out_hbm,
    idx_v, buf0, buf1, buf2,
    g0, g1, g2, w0, w1, w2,
):
    wid = lax.axis_index("s") * NUM_CORES + lax.axis_index("c")
    base = wid * PER_WORKER
    pltpu.sync_copy(idx_hbm.at[pl.ds(base, PER_WORKER)], idx_v)

    bufs = (buf0, buf1, buf2)
    gsems = (g0, g1, g2)
    wsems = (w0, w1, w2)

    def fire_gather(c, b):
        pltpu.async_copy(
            table_hbm.at[idx_v.at[pl.ds(c * CHUNK, CHUNK)]], bufs[b], gsems[b]
        )

    def wait_gather(c, b):
        pltpu.make_async_copy(
            table_hbm.at[idx_v.at[pl.ds(c * CHUNK, CHUNK)]], bufs[b], gsems[b]
        ).wait()

    def fire_write(c, b):
        pltpu.async_copy(
            bufs[b], out_hbm.at[pl.ds(base + c * CHUNK, CHUNK)], wsems[b]
        )

    def wait_write(c, b):
        pltpu.make_async_copy(
            bufs[b], out_hbm.at[pl.ds(base + c * CHUNK, CHUNK)], wsems[b]
        ).wait()

    fire_gather(0, 0)
    fire_gather(1, 1)

    def group_body(q, carry):
        c0 = NBUF * q
        for b in range(NBUF):
            c = c0 + b
            wait_gather(c, b)
            fire_write(c, b)
            nxt = c + 2
            pn = (b + 2) % NBUF

            @pl.when(nxt >= NBUF)
            def _():
                wait_write(nxt - NBUF, pn)

            fire_gather(nxt, pn)

        return carry

    lax.fori_loop(0, NUM_GROUPS, group_body, 0)

    for c in (NUM_CHUNKS - 2, NUM_CHUNKS - 1):
        b = c % NBUF
        wait_gather(c, b)
        fire_write(c, b)

    for c in (NUM_CHUNKS - 3, NUM_CHUNKS - 2, NUM_CHUNKS - 1):
        wait_write(c, c % NBUF)


def kernel(src_seq, pos_embedding):
    flat_idx = src_seq.reshape(-1).astype(jnp.int32)
    out = _gather_rows(flat_idx, pos_embedding)
    return out.reshape(src_seq.shape + (pos_embedding.shape[1],))


# speedup vs baseline: 1.0189x; 1.0189x over previous
import functools

import jax
import jax.numpy as jnp
from jax import lax
from jax.experimental import pallas as pl
from jax.experimental.pallas import tpu as pltpu
from jax.experimental.pallas import tpu_sc as plsc

D_MODEL = 1024
NUM_IDX = 4 * 8192

NUM_CORES = 2
NUM_SUBCORES = 16
NUM_WORKERS = NUM_CORES * NUM_SUBCORES
PER_WORKER = NUM_IDX // NUM_WORKERS
CHUNK = 16
NUM_CHUNKS = PER_WORKER // CHUNK
NBUF = 7
AHEAD = 6
NUM_GROUPS = 9

_mesh = plsc.VectorSubcoreMesh(core_axis_name="c", subcore_axis_name="s")


@functools.partial(
    pl.kernel,
    mesh=_mesh,
    out_type=jax.ShapeDtypeStruct((NUM_IDX, D_MODEL), jnp.float32),
    scratch_types=[
        pltpu.VMEM((PER_WORKER,), jnp.int32),
    ]
    + [pltpu.VMEM((CHUNK, D_MODEL), jnp.float32)] * 7
    + [pltpu.SemaphoreType.DMA] * 14,
)
def _gather_rows(
    idx_hbm, table_hbm, out_hbm,
    idx_v, buf0, buf1, buf2, buf3, buf4, buf5, buf6,
    g0, g1, g2, g3, g4, g5, g6,
    w0, w1, w2, w3, w4, w5, w6,
):
    wid = lax.axis_index("s") * NUM_CORES + lax.axis_index("c")
    base = wid * PER_WORKER
    pltpu.sync_copy(idx_hbm.at[pl.ds(base, PER_WORKER)], idx_v)

    bufs = (buf0, buf1, buf2, buf3, buf4, buf5, buf6)
    gsems = (g0, g1, g2, g3, g4, g5, g6)
    wsems = (w0, w1, w2, w3, w4, w5, w6)

    def fire_gather(c, b):
        pltpu.async_copy(
            table_hbm.at[idx_v.at[pl.ds(c * CHUNK, CHUNK)]], bufs[b], gsems[b]
        )

    def wait_gather(c, b):
        pltpu.make_async_copy(
            table_hbm.at[idx_v.at[pl.ds(c * CHUNK, CHUNK)]], bufs[b], gsems[b]
        ).wait()

    def fire_write(c, b):
        pltpu.async_copy(
            bufs[b], out_hbm.at[pl.ds(base + c * CHUNK, CHUNK)], wsems[b]
        )

    def wait_write(c, b):
        pltpu.make_async_copy(
            bufs[b], out_hbm.at[pl.ds(base + c * CHUNK, CHUNK)], wsems[b]
        ).wait()

    for c in range(AHEAD):
        fire_gather(c, c)

    def group_body(q, carry):
        c0 = NBUF * q
        for b in range(NBUF):
            c = c0 + b
            wait_gather(c, b)
            fire_write(c, b)
            nxt = c + AHEAD
            pn = (b + AHEAD) % NBUF

            @pl.when(nxt < NUM_CHUNKS)
            def _():
                @pl.when(nxt >= NBUF)
                def _():
                    wait_write(nxt - NBUF, pn)

                fire_gather(nxt, pn)

        return carry

    lax.fori_loop(0, NUM_GROUPS, group_body, 0)

    c = NUM_CHUNKS - 1
    wait_gather(c, c % NBUF)
    fire_write(c, c % NBUF)

    for c in range(NUM_CHUNKS - NBUF, NUM_CHUNKS):
        wait_write(c, c % NBUF)


def kernel(src_seq, pos_embedding):
    flat_idx = src_seq.reshape(-1).astype(jnp.int32)
    out = _gather_rows(flat_idx, pos_embedding)
    return out.reshape(src_seq.shape + (pos_embedding.shape[1],))
